# baseline (speedup 1.0000x reference)
import jax
import jax.numpy as jnp
from jax import lax
from jax.experimental import pallas as pl
from jax.experimental.pallas import tpu as pltpu
from jax.experimental.pallas import tpu_sc as plsc

NC = 2
NS = 16
NW = NC * NS

BATCH = 4096
HIST = 200
VOCAB = 1000
D = 64
PAD = 64
N_IDX = BATCH * HIST
B_PER_W = N_IDX // NW

G_ROWS = 256
N_GROUPS = B_PER_W // G_ROWS
BLK = 16
N_BLK = G_ROWS // BLK


def _body(tokens_hbm, table_hbm, out_hbm, tbl_v, idx_v, rows_v, wsem):
    wid = lax.axis_index("s") * NC + lax.axis_index("c")
    base = wid * B_PER_W
    pltpu.sync_copy(table_hbm, tbl_v.at[:, pl.ds(0, D)])
    pltpu.sync_copy(tokens_hbm.at[pl.ds(base, B_PER_W)], idx_v)
    lane = lax.iota(jnp.int32, BLK)

    def compute(g, half):
        @pl.loop(0, N_BLK)
        def _blk(blk):
            toks = idx_v[pl.ds(g * G_ROWS + blk * BLK, BLK)]
            rdst = lane + (half * G_ROWS + blk * BLK)
            for c in range(D):
                cvec = jnp.full((BLK,), c, jnp.int32)
                v = plsc.load_gather(tbl_v, [toks, cvec])
                plsc.store_scatter(rows_v, [rdst, cvec], v)

    def write(g, half):
        return pltpu.make_async_copy(
            rows_v.at[pl.ds(half * G_ROWS, G_ROWS), pl.ds(0, D)],
            out_hbm.at[pl.ds(base + g * G_ROWS, G_ROWS)],
            wsem.at[half],
        )

    compute(0, 0)
    write(0, 0).start()
    compute(1, 1)
    write(1, 1).start()

    @pl.loop(0, (N_GROUPS - 2) // 2)
    def _pair(p):
        g = 2 * p + 2
        write(g - 2, 0).wait()
        compute(g, 0)
        write(g, 0).start()
        write(g - 1, 1).wait()
        compute(g + 1, 1)
        write(g + 1, 1).start()

    write(N_GROUPS - 2, 0).wait()
    write(N_GROUPS - 1, 1).wait()


def kernel(tokens, table):
    flat = tokens.reshape(N_IDX)
    mesh = plsc.VectorSubcoreMesh(core_axis_name="c", subcore_axis_name="s")
    out = pl.kernel(
        _body,
        out_type=jax.ShapeDtypeStruct((N_IDX, D), jnp.float32),
        mesh=mesh,
        compiler_params=pltpu.CompilerParams(
            use_tc_tiling_on_sc=False, needs_layout_passes=False
        ),
        scratch_types=[
            pltpu.VMEM((VOCAB, PAD), jnp.float32),
            pltpu.VMEM((B_PER_W,), jnp.int32),
            pltpu.VMEM((2 * G_ROWS, PAD), jnp.float32),
            pltpu.SemaphoreType.DMA((2,)),
        ],
    )(flat, table)
    return out.reshape(BATCH, HIST, D)


# speedup vs baseline: 2.8745x; 2.8745x over previous
import jax
import jax.numpy as jnp
from jax import lax
from jax.experimental import pallas as pl
from jax.experimental.pallas import tpu as pltpu
from jax.experimental.pallas import tpu_sc as plsc

NC = 2
NS = 16
NW = NC * NS

BATCH = 4096
HIST = 200
EMBED_DIM = 64
N_IDX = BATCH * HIST
B_PER_W = N_IDX // NW
CHUNK = 512
GROUP = 1
G_ROWS = GROUP * CHUNK
N_GROUPS = B_PER_W // G_ROWS


def _body(tokens_hbm, table_hbm, out_hbm, idx_v, rows_v, gsem, wsem):
    wid = lax.axis_index("s") * NC + lax.axis_index("c")
    base = wid * B_PER_W
    pltpu.sync_copy(tokens_hbm.at[pl.ds(base, B_PER_W)], idx_v)

    def gathers(g, b):
        return [
            pltpu.make_async_copy(
                table_hbm.at[idx_v.at[pl.ds(g * G_ROWS + k * CHUNK, CHUNK)]],
                rows_v.at[b, pl.ds(k * CHUNK, CHUNK)],
                gsem.at[b],
            )
            for k in range(GROUP)
        ]

    def write(g, b):
        return pltpu.make_async_copy(
            rows_v.at[b],
            out_hbm.at[pl.ds(base + g * G_ROWS, G_ROWS)],
            wsem.at[b],
        )

    def step(g, b, first=False, last=False):
        for c in gathers(g, b):
            c.wait()
        if not last:
            if not first:
                write(g - 1, 1 - b).wait()
            for c in gathers(g + 1, 1 - b):
                c.start()
        write(g, b).start()

    for c in gathers(0, 0):
        c.start()
    step(0, 0, first=True)

    @pl.loop(0, (N_GROUPS - 2) // 2)
    def _pair(gg):
        step(2 * gg + 1, 1)
        step(2 * gg + 2, 0)

    step(N_GROUPS - 1, 1, last=True)
    write(N_GROUPS - 2, 0).wait()
    write(N_GROUPS - 1, 1).wait()


def kernel(tokens, table):
    flat = tokens.reshape(N_IDX)
    mesh = plsc.VectorSubcoreMesh(core_axis_name="c", subcore_axis_name="s")
    out = pl.kernel(
        _body,
        out_type=jax.ShapeDtypeStruct((N_IDX, EMBED_DIM), jnp.float32),
        mesh=mesh,
        compiler_params=pltpu.CompilerParams(use_tc_tiling_on_sc=False),
        scratch_types=[
            pltpu.VMEM((B_PER_W,), jnp.int32),
            pltpu.VMEM((2, G_ROWS, EMBED_DIM), jnp.float32),
            pltpu.SemaphoreType.DMA((2,)),
            pltpu.SemaphoreType.DMA((2,)),
        ],
    )(flat, table)
    return out.reshape(BATCH, HIST, EMBED_DIM)


# speedup vs baseline: 3.1130x; 1.0830x over previous
import jax
import jax.numpy as jnp
from jax import lax
from jax.experimental import pallas as pl
from jax.experimental.pallas import tpu as pltpu
from jax.experimental.pallas import tpu_sc as plsc

NC = 2
NS = 16

BATCH = 4096
HIST = 200
VOCAB = 1000
D = 64
N_IDX = BATCH * HIST
N_PER_SC = N_IDX // NC

R_ROWS = 6400
N_ROUNDS = N_PER_SC // R_ROWS
T_ROWS = R_ROWS // NS
NBUF = 3


def _body(tokens_hbm, table_hbm, out_hbm, idx_v, local_v, shared,
          isem, gsem, csem, wsem):
    c = lax.axis_index("c")
    s = lax.axis_index("s")

    def idxload(r):
        return pltpu.make_async_copy(
            tokens_hbm.at[c, r, s],
            idx_v.at[lax.rem(r, 3)],
            isem.at[lax.rem(r, 3)],
        )

    def gather(r):
        return pltpu.make_async_copy(
            table_hbm.at[idx_v.at[lax.rem(r, 3)]],
            local_v.at[lax.rem(r, 2)],
            gsem.at[lax.rem(r, 2)],
        )

    def copy(r):
        return pltpu.make_async_copy(
            local_v.at[lax.rem(r, 2)],
            shared.at[lax.rem(r, NBUF), pl.ds(s * T_ROWS, T_ROWS)],
            csem,
        )

    def write(r):
        return pltpu.make_async_copy(
            shared.at[lax.rem(r, NBUF)],
            out_hbm.at[pl.ds((c * N_ROUNDS + r) * R_ROWS, R_ROWS)],
            wsem,
        )

    idxload(0).start()
    idxload(1).start()
    idxload(0).wait()
    gather(0).start()

    @pl.loop(0, N_ROUNDS)
    def _round(r):
        @pl.when(jnp.logical_and(r >= NBUF - 1,
                                 s == lax.rem(r - (NBUF - 1), NS)))
        def _():
            write(r - (NBUF - 1)).wait()

        plsc.subcore_barrier()

        @pl.when(r + 2 < N_ROUNDS)
        def _():
            idxload(r + 2).start()

        @pl.when(r + 1 < N_ROUNDS)
        def _():
            idxload(r + 1).wait()
            gather(r + 1).start()

        gather(r).wait()
        copy(r).start()
        copy(r).wait()
        plsc.subcore_barrier()

        @pl.when(s == lax.rem(r, NS))
        def _():
            write(r).start()

    for r in range(N_ROUNDS - (NBUF - 1), N_ROUNDS):
        @pl.when(s == lax.rem(jnp.int32(r), NS))
        def _():
            write(r).wait()


def kernel(tokens, table):
    idx4 = tokens.reshape(NC, N_ROUNDS, NS, T_ROWS)
    mesh = plsc.VectorSubcoreMesh(core_axis_name="c", subcore_axis_name="s")
    out = pl.kernel(
        _body,
        out_type=jax.ShapeDtypeStruct((N_IDX, D), jnp.float32),
        mesh=mesh,
        compiler_params=pltpu.CompilerParams(use_tc_tiling_on_sc=False),
        scratch_types=[
            pltpu.VMEM((3, T_ROWS), jnp.int32),
            pltpu.VMEM((2, T_ROWS, D), jnp.float32),
            pltpu.VMEM_SHARED((NBUF, R_ROWS, D), jnp.float32),
            pltpu.SemaphoreType.DMA((3,)),
            pltpu.SemaphoreType.DMA((2,)),
            pltpu.SemaphoreType.DMA,
            pltpu.SemaphoreType.DMA,
        ],
    )(idx4, table)
    return out.reshape(BATCH, HIST, D)
